# Initial kernel scaffold; baseline (speedup 1.0000x reference)
import jax
import jax.numpy as jnp
from jax.experimental import pallas as pl


def kernel(x, conv_w, conv_b, bn_g, bn_b, bn_m, bn_v, in_proj_w, in_proj_b, out_proj_w, out_proj_b, ln1_g, ln1_b, ff1_w, ff1_b, ff2_w, ff2_b, ln2_g, ln2_b):
    raise NotImplementedError("write your pallas kernel here")


# speedup vs baseline: 2.0203x; 2.0203x over previous
import functools

import jax
import jax.numpy as jnp
from jax.experimental import pallas as pl
from jax.experimental.pallas import tpu as pltpu

B = 8
C = 96
HW = 110
NPIX = HW * HW
NPAD = 12160
K = 768
NH = 8
DH = C // NH
DFF = 2 * C
CHUNK = 2432
NCHUNK = NPAD // CHUNK
INV_SQRT_DH = 1.0 / (DH ** 0.5)


def _chunk_widths():
    ws = []
    for i in range(NCHUNK):
        start = i * CHUNK
        ws.append(min(CHUNK, NPIX - start))
    return ws


def _cumsum_lanes(v):
    lane = jax.lax.broadcasted_iota(jnp.int32, (1, NPAD), 1)
    n = 1
    while n < NPAD:
        shifted = jnp.roll(v, n, axis=1)
        v = v + jnp.where(lane >= n, shifted, 0)
        n *= 2
    return v


def _layernorm(t, g, b):
    m = jnp.sum(t, axis=1, keepdims=True) * (1.0 / C)
    d = t - m
    var = jnp.sum(d * d, axis=1, keepdims=True) * (1.0 / C)
    return d * jax.lax.rsqrt(var + 1e-5) * g + b


def _dot(a, b, dims):
    return jax.lax.dot_general(a, b, (dims, ((), ())),
                               preferred_element_type=jnp.float32)


def _body(x_ref, pet_ref, w49_ref, aff_ref,
          wi_ref, bi_ref, wo_ref, bo_ref,
          ln1g_ref, ln1b_ref, ff1_ref, f1b_ref, ff2_ref, f2b_ref,
          ln2g_ref, ln2b_ref, out_ref):
    xb = x_ref[0]
    lane = jax.lax.broadcasted_iota(jnp.int32, (1, NPAD), 1)
    rr = lane // HW
    cc = lane - rr * HW
    in_pix = lane < NPIX

    y = _dot(w49_ref[...], xb, ((1,), (0,)))
    y = jnp.concatenate(
        [y, jnp.zeros((49, NPAD - NPIX), jnp.float32)], axis=1)
    acc = jnp.zeros((1, NPAD), jnp.float32)
    for k in range(49):
        p, q = k // 7, k % 7
        dr, dc = 5 * p - 15, 5 * q - 15
        off = HW * dr + dc
        row = y[k:k + 1, :]
        if off != 0:
            row = jnp.roll(row, -off, axis=1)
        valid = ((rr + dr >= 0) & (rr + dr < HW) &
                 (cc + dc >= 0) & (cc + dc < HW) & in_pix)
        acc = acc + jnp.where(valid, row, 0.0)
    shift = aff_ref[0:1, 0:1]
    z = acc + shift
    g = 0.5 * z * (1.0 + jax.lax.erf(z * (2.0 ** -0.5)))
    fmv = 1.0 / (1.0 + jnp.exp(-g))
    scores = jnp.where(in_pix, fmv, -1.0)

    sb = jax.lax.bitcast_convert_type(scores, jnp.int32)

    def _step(_, lohi):
        lo, hi = lohi
        mid = jax.lax.shift_right_logical(lo + hi + 1, 1)
        cnt = jnp.sum(jnp.where(sb >= mid, 1, 0))
        take = cnt >= K
        return jnp.where(take, mid, lo), jnp.where(take, hi, mid - 1)

    lo, _ = jax.lax.fori_loop(0, 31, _step, (jnp.int32(0), jnp.int32(0x3F800000)))
    mask_gt = sb > lo
    mask_eq = sb == lo
    ngt = jnp.sum(jnp.where(mask_gt, 1, 0))
    extra = K - ngt
    cs_eq = _cumsum_lanes(mask_eq.astype(jnp.int32))
    mask = mask_gt | (mask_eq & (cs_eq <= extra))
    incl = _cumsum_lanes(mask.astype(jnp.int32))
    rank = jnp.where(mask, incl - 1, -1)

    widths = _chunk_widths()
    tokens = jnp.zeros((K, C), jnp.float32)
    for i, w in enumerate(widths):
        s = i * CHUNK
        xc = xb[:, s:s + w]
        fmc = fmv[:, s:s + w]
        xsp = xc * fmc + pet_ref[:, s:s + w]
        oh = (jax.lax.broadcasted_iota(jnp.int32, (K, w), 0)
              == rank[:, s:s + w]).astype(jnp.float32)
        tokens = tokens + _dot(oh, xsp, ((1,), (1,)))

    qkv = _dot(tokens, wi_ref[...], ((1,), (0,))) + bi_ref[...]
    ohs = []
    for h in range(NH):
        qh = qkv[:, h * DH:(h + 1) * DH]
        kh = qkv[:, C + h * DH:C + (h + 1) * DH]
        vh = qkv[:, 2 * C + h * DH:2 * C + (h + 1) * DH]
        att = _dot(qh, kh, ((1,), (1,))) * INV_SQRT_DH
        att = att - jnp.max(att, axis=1, keepdims=True)
        e = jnp.exp(att)
        att = e / jnp.sum(e, axis=1, keepdims=True)
        ohs.append(_dot(att, vh, ((1,), (0,))))
    o = jnp.concatenate(ohs, axis=1)
    a = _dot(o, wo_ref[...], ((1,), (0,))) + bo_ref[...]
    t1 = _layernorm(tokens + a, ln1g_ref[...], ln1b_ref[...])
    f = jnp.maximum(_dot(t1, ff1_ref[...], ((1,), (0,))) + f1b_ref[...], 0.0)
    f = _dot(f, ff2_ref[...], ((1,), (0,))) + f2b_ref[...]
    t2 = _layernorm(t1 + f, ln2g_ref[...], ln2b_ref[...])

    for i, w in enumerate(widths):
        s = i * CHUNK
        xc = xb[:, s:s + w]
        fmc = fmv[:, s:s + w]
        oh = (jax.lax.broadcasted_iota(jnp.int32, (K, w), 0)
              == rank[:, s:s + w]).astype(jnp.float32)
        scat = _dot(t2, oh, ((0,), (0,)))
        sel = rank[:, s:s + w] >= 0
        out_ref[0, :, s:s + w] = jnp.where(sel, scat, xc * fmc)


@jax.jit
def kernel(x, conv_w, conv_b, bn_g, bn_b, bn_m, bn_v,
           in_proj_w, in_proj_b, out_proj_w, out_proj_b,
           ln1_g, ln1_b, ff1_w, ff1_b, ff2_w, ff2_b, ln2_g, ln2_b):
    xf = x.reshape(B, C, NPIX)
    scale = bn_g[0] / jnp.sqrt(bn_v[0] + 1e-5)
    shift = (conv_b[0] - bn_m[0]) * scale + bn_b[0]
    w49 = conv_w[0].reshape(C, 49).T * scale
    aff = jnp.full((8, 128), shift, jnp.float32)

    position = jnp.arange(NPIX, dtype=jnp.float32)[None, :]
    half = jnp.exp(jnp.arange(0, C, 2, dtype=jnp.float32)
                   * -(jnp.log(10000.0) / C))
    ang = half[:, None] * position
    pet = jnp.zeros((C, NPIX), jnp.float32)
    pet = pet.at[0::2, :].set(jnp.sin(ang))
    pet = pet.at[1::2, :].set(jnp.cos(ang))

    row = lambda v: v.reshape(1, -1)
    out = pl.pallas_call(
        _body,
        grid=(B,),
        in_specs=[
            pl.BlockSpec((1, C, NPIX), lambda b: (b, 0, 0)),
            pl.BlockSpec((C, NPIX), lambda b: (0, 0)),
            pl.BlockSpec((49, C), lambda b: (0, 0)),
            pl.BlockSpec((8, 128), lambda b: (0, 0)),
            pl.BlockSpec((C, 3 * C), lambda b: (0, 0)),
            pl.BlockSpec((1, 3 * C), lambda b: (0, 0)),
            pl.BlockSpec((C, C), lambda b: (0, 0)),
            pl.BlockSpec((1, C), lambda b: (0, 0)),
            pl.BlockSpec((1, C), lambda b: (0, 0)),
            pl.BlockSpec((1, C), lambda b: (0, 0)),
            pl.BlockSpec((C, DFF), lambda b: (0, 0)),
            pl.BlockSpec((1, DFF), lambda b: (0, 0)),
            pl.BlockSpec((DFF, C), lambda b: (0, 0)),
            pl.BlockSpec((1, C), lambda b: (0, 0)),
            pl.BlockSpec((1, C), lambda b: (0, 0)),
            pl.BlockSpec((1, C), lambda b: (0, 0)),
        ],
        out_specs=pl.BlockSpec((1, C, NPIX), lambda b: (b, 0, 0)),
        out_shape=jax.ShapeDtypeStruct((B, C, NPIX), jnp.float32),
    )(xf, pet, w49, aff,
      in_proj_w.T, row(in_proj_b), out_proj_w.T, row(out_proj_b),
      row(ln1_g), row(ln1_b), ff1_w.T, row(ff1_b), ff2_w.T, row(ff2_b),
      row(ln2_g), row(ln2_b))
    return out.reshape(B, C, HW, HW)


# speedup vs baseline: 2.1996x; 1.0887x over previous
import functools

import jax
import jax.numpy as jnp
import numpy as np
from jax.experimental import pallas as pl
from jax.experimental.pallas import tpu as pltpu

B = 8
C = 96
HW = 110
NPIX = HW * HW
NPAD = 12160
K = 768
NH = 8
DH = C // NH
DFF = 2 * C
CHUNK = 2432
NCHUNK = NPAD // CHUNK
INV_SQRT_DH = 1.0 / (DH ** 0.5)


def _pos_enc_t():
    position = np.arange(NPIX, dtype=np.float32)[None, :]
    half = np.exp(np.arange(0, C, 2, dtype=np.float32)
                  * -(np.log(np.float32(10000.0)) / np.float32(C)))
    ang = (half[:, None] * position).astype(np.float32)
    pet = np.zeros((C, NPIX), np.float32)
    pet[0::2, :] = np.sin(ang)
    pet[1::2, :] = np.cos(ang)
    return pet


_PET = _pos_enc_t()


def _chunk_widths():
    ws = []
    for i in range(NCHUNK):
        start = i * CHUNK
        ws.append(min(CHUNK, NPIX - start))
    return ws


def _cumsum_pair(a, b):
    nrow = NPAD // 128
    m2 = jnp.concatenate([a, b], axis=1).reshape(2 * nrow, 128)
    li = jax.lax.broadcasted_iota(jnp.int32, (128, 128), 0)
    lj = jax.lax.broadcasted_iota(jnp.int32, (128, 128), 1)
    lt = (li <= lj).astype(jnp.float32)
    rowcs = jax.lax.dot_general(
        m2, lt, ((((1,), (0,)), ((), ()))),
        preferred_element_type=jnp.float32,
        precision=jax.lax.Precision.HIGHEST)
    rowtot = rowcs[:, 127:128]
    pi = jax.lax.broadcasted_iota(jnp.int32, (2 * nrow, 2 * nrow), 0)
    pj = jax.lax.broadcasted_iota(jnp.int32, (2 * nrow, 2 * nrow), 1)
    blk = ((pj < pi) & (pi // nrow == pj // nrow)).astype(jnp.float32)
    rowpre = jax.lax.dot_general(
        blk, rowtot, ((((1,), (0,)), ((), ()))),
        preferred_element_type=jnp.float32,
        precision=jax.lax.Precision.HIGHEST)
    flat = (rowcs + rowpre).reshape(1, 2 * NPAD)
    return flat[:, :NPAD], flat[:, NPAD:]


def _layernorm(t, g, b):
    m = jnp.sum(t, axis=1, keepdims=True) * (1.0 / C)
    d = t - m
    var = jnp.sum(d * d, axis=1, keepdims=True) * (1.0 / C)
    return d * jax.lax.rsqrt(var + 1e-5) * g + b


def _dot(a, b, dims):
    return jax.lax.dot_general(a, b, (dims, ((), ())),
                               preferred_element_type=jnp.float32)


def _body(x_ref, pet_ref, w49_ref, aff_ref,
          wi_ref, bi_ref, wo_ref, bo_ref,
          ln1g_ref, ln1b_ref, ff1_ref, f1b_ref, ff2_ref, f2b_ref,
          ln2g_ref, ln2b_ref, out_ref):
    xb = x_ref[0]
    lane = jax.lax.broadcasted_iota(jnp.int32, (1, NPAD), 1)
    rr = lane // HW
    cc = lane - rr * HW
    in_pix = lane < NPIX

    y = _dot(w49_ref[...], xb, ((1,), (0,)))
    y = jnp.concatenate(
        [y, jnp.zeros((49, NPAD - NPIX), jnp.float32)], axis=1)
    acc = jnp.zeros((1, NPAD), jnp.float32)
    for k in range(49):
        p, q = k // 7, k % 7
        dr, dc = 5 * p - 15, 5 * q - 15
        off = HW * dr + dc
        row = y[k:k + 1, :]
        if off != 0:
            row = jnp.roll(row, -off, axis=1)
        valid = ((rr + dr >= 0) & (rr + dr < HW) &
                 (cc + dc >= 0) & (cc + dc < HW) & in_pix)
        acc = acc + jnp.where(valid, row, 0.0)
    shift = aff_ref[0:1, 0:1]
    z = acc + shift
    g = 0.5 * z * (1.0 + jax.lax.erf(z * (2.0 ** -0.5)))
    fmv = 1.0 / (1.0 + jnp.exp(-g))
    scores = jnp.where(in_pix, fmv, -1.0)

    sb = jax.lax.bitcast_convert_type(scores, jnp.int32)

    def _step(_, lohi):
        lo, hi = lohi
        mid = jax.lax.shift_right_logical(lo + hi + 1, 1)
        cnt = jnp.sum(jnp.where(sb >= mid, 1, 0))
        take = cnt >= K
        return jnp.where(take, mid, lo), jnp.where(take, hi, mid - 1)

    lo, _ = jax.lax.fori_loop(0, 31, _step, (jnp.int32(0), jnp.int32(0x3F800000)))
    mask_gt = sb > lo
    mask_eq = sb == lo
    ngt = jnp.sum(jnp.where(mask_gt, 1, 0))
    extra = K - ngt
    cs_gt_f, cs_eq_f = _cumsum_pair(jnp.where(mask_gt, 1.0, 0.0),
                                    jnp.where(mask_eq, 1.0, 0.0))
    cs_gt = cs_gt_f.astype(jnp.int32)
    cs_eq = cs_eq_f.astype(jnp.int32)
    mask = mask_gt | (mask_eq & (cs_eq <= extra))
    incl = cs_gt + jnp.minimum(cs_eq, extra)
    rank = jnp.where(mask, incl - 1, -1)

    widths = _chunk_widths()
    tokens = jnp.zeros((K, C), jnp.float32)
    for i, w in enumerate(widths):
        s = i * CHUNK
        xc = xb[:, s:s + w]
        fmc = fmv[:, s:s + w]
        xsp = xc * fmc + pet_ref[:, s:s + w]
        oh = (jax.lax.broadcasted_iota(jnp.int32, (K, w), 0)
              == rank[:, s:s + w]).astype(jnp.float32)
        tokens = tokens + _dot(oh, xsp, ((1,), (1,)))

    qkv = _dot(tokens, wi_ref[...], ((1,), (0,))) + bi_ref[...]
    ohs = []
    for h in range(NH):
        qh = qkv[:, h * DH:(h + 1) * DH]
        kh = qkv[:, C + h * DH:C + (h + 1) * DH]
        vh = qkv[:, 2 * C + h * DH:2 * C + (h + 1) * DH]
        att = _dot(qh, kh, ((1,), (1,))) * INV_SQRT_DH
        att = att - jnp.max(att, axis=1, keepdims=True)
        e = jnp.exp(att)
        att = e / jnp.sum(e, axis=1, keepdims=True)
        ohs.append(_dot(att, vh, ((1,), (0,))))
    o = jnp.concatenate(ohs, axis=1)
    a = _dot(o, wo_ref[...], ((1,), (0,))) + bo_ref[...]
    t1 = _layernorm(tokens + a, ln1g_ref[...], ln1b_ref[...])
    f = jnp.maximum(_dot(t1, ff1_ref[...], ((1,), (0,))) + f1b_ref[...], 0.0)
    f = _dot(f, ff2_ref[...], ((1,), (0,))) + f2b_ref[...]
    t2 = _layernorm(t1 + f, ln2g_ref[...], ln2b_ref[...])

    for i, w in enumerate(widths):
        s = i * CHUNK
        xc = xb[:, s:s + w]
        fmc = fmv[:, s:s + w]
        oh = (jax.lax.broadcasted_iota(jnp.int32, (K, w), 0)
              == rank[:, s:s + w]).astype(jnp.float32)
        scat = _dot(t2, oh, ((0,), (0,)))
        sel = rank[:, s:s + w] >= 0
        out_ref[0, :, s:s + w] = jnp.where(sel, scat, xc * fmc)


@jax.jit
def kernel(x, conv_w, conv_b, bn_g, bn_b, bn_m, bn_v,
           in_proj_w, in_proj_b, out_proj_w, out_proj_b,
           ln1_g, ln1_b, ff1_w, ff1_b, ff2_w, ff2_b, ln2_g, ln2_b):
    xf = x.reshape(B, C, NPIX)
    scale = bn_g[0] / jnp.sqrt(bn_v[0] + 1e-5)
    shift = (conv_b[0] - bn_m[0]) * scale + bn_b[0]
    w49 = conv_w[0].reshape(C, 49).T * scale
    aff = jnp.full((8, 128), shift, jnp.float32)

    pet = jnp.asarray(_PET)

    row = lambda v: v.reshape(1, -1)
    out = pl.pallas_call(
        _body,
        grid=(B,),
        in_specs=[
            pl.BlockSpec((1, C, NPIX), lambda b: (b, 0, 0)),
            pl.BlockSpec((C, NPIX), lambda b: (0, 0)),
            pl.BlockSpec((49, C), lambda b: (0, 0)),
            pl.BlockSpec((8, 128), lambda b: (0, 0)),
            pl.BlockSpec((C, 3 * C), lambda b: (0, 0)),
            pl.BlockSpec((1, 3 * C), lambda b: (0, 0)),
            pl.BlockSpec((C, C), lambda b: (0, 0)),
            pl.BlockSpec((1, C), lambda b: (0, 0)),
            pl.BlockSpec((1, C), lambda b: (0, 0)),
            pl.BlockSpec((1, C), lambda b: (0, 0)),
            pl.BlockSpec((C, DFF), lambda b: (0, 0)),
            pl.BlockSpec((1, DFF), lambda b: (0, 0)),
            pl.BlockSpec((DFF, C), lambda b: (0, 0)),
            pl.BlockSpec((1, C), lambda b: (0, 0)),
            pl.BlockSpec((1, C), lambda b: (0, 0)),
            pl.BlockSpec((1, C), lambda b: (0, 0)),
        ],
        out_specs=pl.BlockSpec((1, C, NPIX), lambda b: (b, 0, 0)),
        out_shape=jax.ShapeDtypeStruct((B, C, NPIX), jnp.float32),
    )(xf, pet, w49, aff,
      in_proj_w.T, row(in_proj_b), out_proj_w.T, row(out_proj_b),
      row(ln1_g), row(ln1_b), ff1_w.T, row(ff1_b), ff2_w.T, row(ff2_b),
      row(ln2_g), row(ln2_b))
    return out.reshape(B, C, HW, HW)


# speedup vs baseline: 2.3498x; 1.0683x over previous
import functools

import jax
import jax.numpy as jnp
import numpy as np
from jax.experimental import pallas as pl
from jax.experimental.pallas import tpu as pltpu

B = 8
C = 96
HW = 110
LROW = 128
LCAN = HW * LROW
PADL = 2048
K = 768
NH = 8
DH = C // NH
DFF = 2 * C
NCHUNK = 5
CHUNK = LCAN // NCHUNK
ROWS_PER_CHUNK = HW // NCHUNK
INV_SQRT_DH = 1.0 / (DH ** 0.5)


def _pos_enc_canvas():
    position = np.arange(HW * HW, dtype=np.float32)[None, :]
    half = np.exp(np.arange(0, C, 2, dtype=np.float32)
                  * -(np.log(np.float32(10000.0)) / np.float32(C)))
    ang = (half[:, None] * position).astype(np.float32)
    pet = np.zeros((C, HW * HW), np.float32)
    pet[0::2, :] = np.sin(ang)
    pet[1::2, :] = np.cos(ang)
    out = np.zeros((C, HW, LROW), np.float32)
    out[:, :, :HW] = pet.reshape(C, HW, HW)
    return out.reshape(C, LCAN)


_PET = _pos_enc_canvas()


def _cumsum_pair(a, b):
    nrow = LCAN // 128
    m2 = jnp.concatenate([a, b], axis=1).reshape(2 * nrow, 128)
    li = jax.lax.broadcasted_iota(jnp.int32, (128, 128), 0)
    lj = jax.lax.broadcasted_iota(jnp.int32, (128, 128), 1)
    lt = (li <= lj).astype(jnp.float32)
    rowcs = jax.lax.dot_general(
        m2, lt, ((((1,), (0,)), ((), ()))),
        preferred_element_type=jnp.float32,
        precision=jax.lax.Precision.HIGHEST)
    rowtot = rowcs[:, 127:128]
    pi = jax.lax.broadcasted_iota(jnp.int32, (2 * nrow, 2 * nrow), 0)
    pj = jax.lax.broadcasted_iota(jnp.int32, (2 * nrow, 2 * nrow), 1)
    blk = ((pj < pi) & (pi // nrow == pj // nrow)).astype(jnp.float32)
    rowpre = jax.lax.dot_general(
        blk, rowtot, ((((1,), (0,)), ((), ()))),
        preferred_element_type=jnp.float32,
        precision=jax.lax.Precision.HIGHEST)
    flat = (rowcs + rowpre).reshape(1, 2 * LCAN)
    return flat[:, :LCAN], flat[:, LCAN:]


def _layernorm(t, g, b):
    m = jnp.sum(t, axis=1, keepdims=True) * (1.0 / C)
    d = t - m
    var = jnp.sum(d * d, axis=1, keepdims=True) * (1.0 / C)
    return d * jax.lax.rsqrt(var + 1e-5) * g + b


def _dot(a, b, dims):
    return jax.lax.dot_general(a, b, (dims, ((), ())),
                               preferred_element_type=jnp.float32)


def _body(x_ref, pet_ref, w49_ref, aff_ref,
          wi_ref, bi_ref, wo_ref, bo_ref,
          ln1g_ref, ln1b_ref, ff1_ref, f1b_ref, ff2_ref, f2b_ref,
          ln2g_ref, ln2b_ref, out_ref):
    x3 = x_ref[0]
    xcan = jnp.concatenate(
        [x3, jnp.zeros((C, HW, LROW - HW), jnp.float32)],
        axis=2).reshape(C, LCAN)
    lane = jax.lax.broadcasted_iota(jnp.int32, (1, LCAN), 1)
    in_pix = (lane & 127) < HW

    y = _dot(w49_ref[...], xcan, ((1,), (0,)))
    ycat = jnp.concatenate(
        [jnp.zeros((49, PADL), jnp.float32), y,
         jnp.zeros((49, PADL), jnp.float32)], axis=1)
    acc = jnp.zeros((1, LCAN), jnp.float32)
    for k in range(49):
        p, q = k // 7, k % 7
        off = LROW * (5 * p - 15) + (5 * q - 15)
        acc = acc + ycat[k:k + 1, PADL + off:PADL + off + LCAN]
    shift = aff_ref[0:1, 0:1]
    z = acc + shift
    g = 0.5 * z * (1.0 + jax.lax.erf(z * (2.0 ** -0.5)))
    fmv = 1.0 / (1.0 + jnp.exp(-g))
    scores = jnp.where(in_pix, fmv, -1.0)

    sb = jax.lax.bitcast_convert_type(scores, jnp.int32)

    def _step(_, lohi):
        lo, hi = lohi
        mid = jax.lax.shift_right_logical(lo + hi + 1, 1)
        cnt = jnp.sum(jnp.where(sb >= mid, 1, 0))
        take = cnt >= K
        return jnp.where(take, mid, lo), jnp.where(take, hi, mid - 1)

    lo, _ = jax.lax.fori_loop(0, 31, _step, (jnp.int32(0), jnp.int32(0x3F800000)))
    mask_gt = sb > lo
    mask_eq = sb == lo
    ngt = jnp.sum(jnp.where(mask_gt, 1, 0))
    extra = K - ngt
    cs_gt_f, cs_eq_f = _cumsum_pair(jnp.where(mask_gt, 1.0, 0.0),
                                    jnp.where(mask_eq, 1.0, 0.0))
    cs_gt = cs_gt_f.astype(jnp.int32)
    cs_eq = cs_eq_f.astype(jnp.int32)
    mask = mask_gt | (mask_eq & (cs_eq <= extra))
    incl = cs_gt + jnp.minimum(cs_eq, extra)
    rank = jnp.where(mask, incl - 1, -1)

    tokens = jnp.zeros((K, C), jnp.float32)
    for i in range(NCHUNK):
        s = i * CHUNK
        xc = xcan[:, s:s + CHUNK]
        fmc = fmv[:, s:s + CHUNK]
        xsp = (xc * fmc + pet_ref[:, s:s + CHUNK]).astype(jnp.bfloat16)
        oh = (jax.lax.broadcasted_iota(jnp.int32, (K, CHUNK), 0)
              == rank[:, s:s + CHUNK]).astype(jnp.bfloat16)
        tokens = tokens + _dot(oh, xsp, ((1,), (1,)))

    qkv = _dot(tokens, wi_ref[...], ((1,), (0,))) + bi_ref[...]
    ohs = []
    for h in range(NH):
        qh = qkv[:, h * DH:(h + 1) * DH]
        kh = qkv[:, C + h * DH:C + (h + 1) * DH]
        vh = qkv[:, 2 * C + h * DH:2 * C + (h + 1) * DH]
        att = _dot(qh, kh, ((1,), (1,))) * INV_SQRT_DH
        att = att - jnp.max(att, axis=1, keepdims=True)
        e = jnp.exp(att)
        att = e / jnp.sum(e, axis=1, keepdims=True)
        ohs.append(_dot(att, vh, ((1,), (0,))))
    o = jnp.concatenate(ohs, axis=1)
    a = _dot(o, wo_ref[...], ((1,), (0,))) + bo_ref[...]
    t1 = _layernorm(tokens + a, ln1g_ref[...], ln1b_ref[...])
    f = jnp.maximum(_dot(t1, ff1_ref[...], ((1,), (0,))) + f1b_ref[...], 0.0)
    f = _dot(f, ff2_ref[...], ((1,), (0,))) + f2b_ref[...]
    t2 = _layernorm(t1 + f, ln2g_ref[...], ln2b_ref[...])
    t2b = t2.astype(jnp.bfloat16)

    for i in range(NCHUNK):
        s = i * CHUNK
        xc = xcan[:, s:s + CHUNK]
        fmc = fmv[:, s:s + CHUNK]
        oh = (jax.lax.broadcasted_iota(jnp.int32, (K, CHUNK), 0)
              == rank[:, s:s + CHUNK]).astype(jnp.bfloat16)
        scat = _dot(t2b, oh, ((0,), (0,)))
        sel = rank[:, s:s + CHUNK] >= 0
        res = jnp.where(sel, scat, xc * fmc)
        res3 = res.reshape(C, ROWS_PER_CHUNK, LROW)[:, :, :HW]
        out_ref[0, :, i * ROWS_PER_CHUNK:(i + 1) * ROWS_PER_CHUNK, :] = res3


@jax.jit
def kernel(x, conv_w, conv_b, bn_g, bn_b, bn_m, bn_v,
           in_proj_w, in_proj_b, out_proj_w, out_proj_b,
           ln1_g, ln1_b, ff1_w, ff1_b, ff2_w, ff2_b, ln2_g, ln2_b):
    scale = bn_g[0] / jnp.sqrt(bn_v[0] + 1e-5)
    shift = (conv_b[0] - bn_m[0]) * scale + bn_b[0]
    w49 = conv_w[0].reshape(C, 49).T * scale
    aff = jnp.full((8, 128), shift, jnp.float32)
    pet = jnp.asarray(_PET)

    row = lambda v: v.reshape(1, -1)
    out = pl.pallas_call(
        _body,
        grid=(B,),
        in_specs=[
            pl.BlockSpec((1, C, HW, HW), lambda b: (b, 0, 0, 0)),
            pl.BlockSpec((C, LCAN), lambda b: (0, 0)),
            pl.BlockSpec((49, C), lambda b: (0, 0)),
            pl.BlockSpec((8, 128), lambda b: (0, 0)),
            pl.BlockSpec((C, 3 * C), lambda b: (0, 0)),
            pl.BlockSpec((1, 3 * C), lambda b: (0, 0)),
            pl.BlockSpec((C, C), lambda b: (0, 0)),
            pl.BlockSpec((1, C), lambda b: (0, 0)),
            pl.BlockSpec((1, C), lambda b: (0, 0)),
            pl.BlockSpec((1, C), lambda b: (0, 0)),
            pl.BlockSpec((C, DFF), lambda b: (0, 0)),
            pl.BlockSpec((1, DFF), lambda b: (0, 0)),
            pl.BlockSpec((DFF, C), lambda b: (0, 0)),
            pl.BlockSpec((1, C), lambda b: (0, 0)),
            pl.BlockSpec((1, C), lambda b: (0, 0)),
            pl.BlockSpec((1, C), lambda b: (0, 0)),
        ],
        out_specs=pl.BlockSpec((1, C, HW, HW), lambda b: (b, 0, 0, 0)),
        out_shape=jax.ShapeDtypeStruct((B, C, HW, HW), jnp.float32),
    )(x, pet, w49, aff,
      in_proj_w.T, row(in_proj_b), out_proj_w.T, row(out_proj_b),
      row(ln1_g), row(ln1_b), ff1_w.T, row(ff1_b), ff2_w.T, row(ff2_b),
      row(ln2_g), row(ln2_b))
    return out
